# Initial kernel scaffold; baseline (speedup 1.0000x reference)
import jax
import jax.numpy as jnp
from jax.experimental import pallas as pl


def kernel(text, emb_table, W1, b1, W2, b2):
    raise NotImplementedError("write your pallas kernel here")


# speedup vs baseline: 2.5118x; 2.5118x over previous
import functools

import jax
import jax.numpy as jnp
from jax import lax
from jax.experimental import pallas as pl
from jax.experimental.pallas import tpu as pltpu
from jax.experimental.pallas import tpu_sc as plsc

NUM_EMB = 1000000
TEXT_DIM = 64
OUT_DIM = 128
BATCH = 16384
HIST = 50

NC = 2
NS = 16
NW = NC * NS
ROWS_PER_W = BATCH // NW
K = 8
CHUNK_IDX = 2 * HIST
GROUP_ROWS = 2 * K
NGROUPS = ROWS_PER_W // GROUP_ROWS
NL = TEXT_DIM // 16


def _gather_pool_body(text2_hbm, table_hbm, out_hbm, idx_v, rows_v, stage_v, sem):
    wid = lax.axis_index("s") * NC + lax.axis_index("c")

    def group(g, carry):
        chunk_base = wid * (ROWS_PER_W // 2) + g * K
        pltpu.sync_copy(text2_hbm.at[pl.ds(chunk_base, K)], idx_v)
        copies = [
            pltpu.async_copy(table_hbm.at[idx_v.at[j]], rows_v.at[j], sem)
            for j in range(K)
        ]
        for c in copies:
            c.wait()
        for j in range(K):
            def red(r, accs, j=j):
                lo = tuple(accs[c] + rows_v[j, r, pl.ds(16 * c, 16)]
                           for c in range(NL))
                hi = tuple(accs[NL + c] + rows_v[j, HIST + r, pl.ds(16 * c, 16)]
                           for c in range(NL))
                return lo + hi

            zero = tuple(jnp.zeros((16,), jnp.float32) for _ in range(2 * NL))
            accs = lax.fori_loop(0, HIST, red, zero)
            for c in range(NL):
                stage_v[2 * j, pl.ds(16 * c, 16)] = accs[c]
                stage_v[2 * j + 1, pl.ds(16 * c, 16)] = accs[NL + c]
        pltpu.sync_copy(
            stage_v,
            out_hbm.at[pl.ds(wid * ROWS_PER_W + g * GROUP_ROWS, GROUP_ROWS)])
        return carry

    lax.fori_loop(0, NGROUPS, group, 0)


@functools.cache
def _gather_pool():
    return pl.kernel(
        _gather_pool_body,
        out_type=jax.ShapeDtypeStruct((BATCH, TEXT_DIM), jnp.float32),
        mesh=plsc.VectorSubcoreMesh(core_axis_name="c", subcore_axis_name="s"),
        compiler_params=pltpu.CompilerParams(use_tc_tiling_on_sc=False),
        scratch_types=[
            pltpu.VMEM((K, CHUNK_IDX), jnp.int32),
            pltpu.VMEM((K, CHUNK_IDX, TEXT_DIM), jnp.float32),
            pltpu.VMEM((GROUP_ROWS, TEXT_DIM), jnp.float32),
            pltpu.SemaphoreType.DMA,
        ],
    )


TB = 1024


def _mlp_body(text_ref, summed_ref, w1_ref, b1_ref, w2_ref, b2_ref, out_ref):
    t = text_ref[...]
    counts = jnp.sum((t != 0).astype(jnp.float32), axis=1, keepdims=True)
    pooled = summed_ref[...] / jnp.maximum(counts, 1.0)
    h = jnp.maximum(
        jnp.dot(pooled, w1_ref[...], preferred_element_type=jnp.float32)
        + b1_ref[...], 0.0)
    out_ref[...] = (
        jnp.dot(h, w2_ref[...], preferred_element_type=jnp.float32)
        + b2_ref[...])


_mlp = pl.pallas_call(
    _mlp_body,
    grid=(BATCH // TB,),
    in_specs=[
        pl.BlockSpec((TB, HIST), lambda i: (i, 0)),
        pl.BlockSpec((TB, TEXT_DIM), lambda i: (i, 0)),
        pl.BlockSpec((TEXT_DIM, OUT_DIM), lambda i: (0, 0)),
        pl.BlockSpec((1, OUT_DIM), lambda i: (0, 0)),
        pl.BlockSpec((OUT_DIM, OUT_DIM), lambda i: (0, 0)),
        pl.BlockSpec((1, OUT_DIM), lambda i: (0, 0)),
    ],
    out_specs=pl.BlockSpec((TB, OUT_DIM), lambda i: (i, 0)),
    out_shape=jax.ShapeDtypeStruct((BATCH, OUT_DIM), jnp.float32),
)


def kernel(text, emb_table, W1, b1, W2, b2):
    text = text.astype(jnp.int32)
    text2 = text.reshape(BATCH // 2, CHUNK_IDX)
    summed = _gather_pool()(text2, emb_table)
    return _mlp(text, summed, W1, b1.reshape(1, OUT_DIM),
                W2, b2.reshape(1, OUT_DIM))


# speedup vs baseline: 2.7602x; 1.0989x over previous
import functools

import jax
import jax.numpy as jnp
from jax import lax
from jax.experimental import pallas as pl
from jax.experimental.pallas import tpu as pltpu
from jax.experimental.pallas import tpu_sc as plsc

NUM_EMB = 1000000
TEXT_DIM = 64
OUT_DIM = 128
BATCH = 16384
HIST = 50

NC = 2
NS = 16
NW = NC * NS
ROWS_PER_W = BATCH // NW
K = 8
CHUNK_IDX = 2 * HIST
GROUP_ROWS = 2 * K
NGROUPS = ROWS_PER_W // GROUP_ROWS
NL = TEXT_DIM // 16


def _gather_pool_body(text2_hbm, table_hbm, out_hbm,
                      idx0, idx1, rows0, rows1, stage_v, sem0, sem1):
    wid = lax.axis_index("s") * NC + lax.axis_index("c")

    def fire(g, ib, rb, sem):
        chunk_base = wid * (ROWS_PER_W // 2) + g * K
        pltpu.sync_copy(text2_hbm.at[pl.ds(chunk_base, K)], ib)
        for j in range(K):
            pltpu.async_copy(table_hbm.at[ib.at[j]], rb.at[j], sem)

    def drain_reduce_store(g, ib, rb, sem):
        for j in range(K):
            pltpu.make_async_copy(table_hbm.at[ib.at[j]], rb.at[j], sem).wait()
        for j in range(K):
            def red(r, accs, j=j):
                lo = tuple(accs[c] + rb[j, r, pl.ds(16 * c, 16)]
                           for c in range(NL))
                hi = tuple(accs[NL + c] + rb[j, HIST + r, pl.ds(16 * c, 16)]
                           for c in range(NL))
                return lo + hi

            zero = tuple(jnp.zeros((16,), jnp.float32) for _ in range(2 * NL))
            accs = lax.fori_loop(0, HIST, red, zero)
            for c in range(NL):
                stage_v[2 * j, pl.ds(16 * c, 16)] = accs[c]
                stage_v[2 * j + 1, pl.ds(16 * c, 16)] = accs[NL + c]
        pltpu.sync_copy(
            stage_v,
            out_hbm.at[pl.ds(wid * ROWS_PER_W + g * GROUP_ROWS, GROUP_ROWS)])

    fire(0, idx0, rows0, sem0)

    def body(t, carry):
        g = 2 * t
        fire(g + 1, idx1, rows1, sem1)
        drain_reduce_store(g, idx0, rows0, sem0)
        fire(g + 2, idx0, rows0, sem0)
        drain_reduce_store(g + 1, idx1, rows1, sem1)
        return carry

    lax.fori_loop(0, NGROUPS // 2 - 1, body, 0)
    fire(NGROUPS - 1, idx1, rows1, sem1)
    drain_reduce_store(NGROUPS - 2, idx0, rows0, sem0)
    drain_reduce_store(NGROUPS - 1, idx1, rows1, sem1)


@functools.cache
def _gather_pool():
    return pl.kernel(
        _gather_pool_body,
        out_type=jax.ShapeDtypeStruct((BATCH, TEXT_DIM), jnp.float32),
        mesh=plsc.VectorSubcoreMesh(core_axis_name="c", subcore_axis_name="s"),
        compiler_params=pltpu.CompilerParams(use_tc_tiling_on_sc=False),
        scratch_types=[
            pltpu.VMEM((K, CHUNK_IDX), jnp.int32),
            pltpu.VMEM((K, CHUNK_IDX), jnp.int32),
            pltpu.VMEM((K, CHUNK_IDX, TEXT_DIM), jnp.float32),
            pltpu.VMEM((K, CHUNK_IDX, TEXT_DIM), jnp.float32),
            pltpu.VMEM((GROUP_ROWS, TEXT_DIM), jnp.float32),
            pltpu.SemaphoreType.DMA,
            pltpu.SemaphoreType.DMA,
        ],
    )


TB = 1024


def _mlp_body(text_ref, summed_ref, w1_ref, b1_ref, w2_ref, b2_ref, out_ref):
    t = text_ref[...]
    counts = jnp.sum((t != 0).astype(jnp.float32), axis=1, keepdims=True)
    pooled = summed_ref[...] / jnp.maximum(counts, 1.0)
    h = jnp.maximum(
        jnp.dot(pooled, w1_ref[...], preferred_element_type=jnp.float32)
        + b1_ref[...], 0.0)
    out_ref[...] = (
        jnp.dot(h, w2_ref[...], preferred_element_type=jnp.float32)
        + b2_ref[...])


_mlp = pl.pallas_call(
    _mlp_body,
    grid=(BATCH // TB,),
    in_specs=[
        pl.BlockSpec((TB, HIST), lambda i: (i, 0)),
        pl.BlockSpec((TB, TEXT_DIM), lambda i: (i, 0)),
        pl.BlockSpec((TEXT_DIM, OUT_DIM), lambda i: (0, 0)),
        pl.BlockSpec((1, OUT_DIM), lambda i: (0, 0)),
        pl.BlockSpec((OUT_DIM, OUT_DIM), lambda i: (0, 0)),
        pl.BlockSpec((1, OUT_DIM), lambda i: (0, 0)),
    ],
    out_specs=pl.BlockSpec((TB, OUT_DIM), lambda i: (i, 0)),
    out_shape=jax.ShapeDtypeStruct((BATCH, OUT_DIM), jnp.float32),
)


def kernel(text, emb_table, W1, b1, W2, b2):
    text = text.astype(jnp.int32)
    text2 = text.reshape(BATCH // 2, CHUNK_IDX)
    summed = _gather_pool()(text2, emb_table)
    return _mlp(text, summed, W1, b1.reshape(1, OUT_DIM),
                W2, b2.reshape(1, OUT_DIM))
